# baseline (speedup 1.0000x reference)
import jax
import jax.numpy as jnp
from jax.experimental import pallas as pl
from jax.experimental.pallas import tpu as pltpu

_THR = 0.18
_B, _C, _H, _W = 2, 96, 512, 512
_HR = _H // 2
_BR = 32
_NR = _HR // _BR


def _body(sig_ref, mask_ref, val_ref, p_ref):
    r = pl.program_id(1)
    s = sig_ref[0][:, :, 0:_W]
    hm = jnp.mean(s, axis=0)

    rr = r * _BR + jax.lax.broadcasted_iota(jnp.int32, (_BR, _W), 0)
    col = jax.lax.broadcasted_iota(jnp.int32, (_BR, _W), 1)
    pat1 = jnp.where((col % 4) == 1, 1.0, 0.0)
    pat3 = jnp.where((col % 4) == 3, 1.0, 0.0)
    row_even = jnp.where((rr % 2) == 0, 1.0, 0.0)
    patt = row_even * pat1 + (1.0 - row_even) * pat3
    below = jnp.where(hm <= _THR, 1.0, 0.0)
    p_ref[pl.ds(r * _BR, _BR), :] = below * patt

    @pl.when(r == _NR - 1)
    def _tail():
        pf = p_ref[...]
        colh = jax.lax.broadcasted_iota(jnp.int32, (_HR, _W), 1)
        rowh = jax.lax.broadcasted_iota(jnp.int32, (_HR, _W), 0)
        even_col = jnp.where((colh % 2) == 0, 1.0, 0.0)
        psr = jnp.roll(pf, 1, axis=1) * jnp.where(colh > 0, 1.0, 0.0)
        psl = jnp.roll(pf, -1, axis=1)
        pdn = jnp.roll(pf, -1, axis=0) * jnp.where(rowh < _HR - 1, 1.0, 0.0)

        me = even_col * (1.0 - psr)
        mo = (1.0 - even_col) * (1.0 - jnp.maximum(pf, pdn))
        mask_ref[0, 0, :, 0:_W] = me
        mask_ref[0, 0, :, _W:] = mo
        val_ref[0, 0, :, 0:_W] = psl
        val_ref[0, 0, :, _W:] = jnp.zeros((_HR, _W), jnp.float32)


@jax.jit
def kernel(sigma):
    sig4 = sigma.reshape(_B, _C, _HR, 2 * _W)
    out_sds = jax.ShapeDtypeStruct((_B, 1, _HR, 2 * _W), jnp.float32)
    mask, values = pl.pallas_call(
        _body,
        grid=(_B, _NR),
        in_specs=[pl.BlockSpec((1, _C, _BR, 2 * _W), lambda b, r: (b, 0, r, 0))],
        out_specs=[
            pl.BlockSpec((1, 1, _HR, 2 * _W), lambda b, r: (b, 0, 0, 0)),
            pl.BlockSpec((1, 1, _HR, 2 * _W), lambda b, r: (b, 0, 0, 0)),
        ],
        out_shape=[out_sds, out_sds],
        scratch_shapes=[pltpu.VMEM((_HR, _W), jnp.float32)],
        compiler_params=pltpu.CompilerParams(
            dimension_semantics=("arbitrary", "arbitrary"),
        ),
    )(sig4)
    return mask.reshape(_B, 1, _H, _W), values.reshape(_B, 1, _H, _W)


# speedup vs baseline: 1.1297x; 1.1297x over previous
import jax
import jax.numpy as jnp
from jax.experimental import pallas as pl
from jax.experimental.pallas import tpu as pltpu

_THR = 0.18
_B, _C, _H, _W = 2, 96, 512, 512
_HR = _H // 2
_BR = 32
_NR = _HR // _BR
_NSPLIT = 4
_CS = _C // _NSPLIT


def _body(*refs):
    sig_refs = refs[:_NSPLIT]
    mask_ref, val_ref, p_ref = refs[_NSPLIT:]
    r = pl.program_id(1)
    acc = sig_refs[0][0].astype(jnp.float32)
    for sr in sig_refs[1:]:
        acc = acc + sr[0]
    hm = jnp.sum(acc, axis=0) * (1.0 / _C)

    rr = r * _BR + jax.lax.broadcasted_iota(jnp.int32, (_BR, _W), 0)
    col = jax.lax.broadcasted_iota(jnp.int32, (_BR, _W), 1)
    pat1 = jnp.where((col % 4) == 1, 1.0, 0.0)
    pat3 = jnp.where((col % 4) == 3, 1.0, 0.0)
    row_even = jnp.where((rr % 2) == 0, 1.0, 0.0)
    patt = row_even * pat1 + (1.0 - row_even) * pat3
    below = jnp.where(hm <= _THR, 1.0, 0.0)
    p_ref[pl.ds(r * _BR, _BR), :] = below * patt

    @pl.when(r == _NR - 1)
    def _tail():
        pf = p_ref[...]
        colh = jax.lax.broadcasted_iota(jnp.int32, (_HR, _W), 1)
        rowh = jax.lax.broadcasted_iota(jnp.int32, (_HR, _W), 0)
        even_col = jnp.where((colh % 2) == 0, 1.0, 0.0)
        psr = jnp.roll(pf, 1, axis=1) * jnp.where(colh > 0, 1.0, 0.0)
        psl = jnp.roll(pf, -1, axis=1)
        pdn = jnp.roll(pf, -1, axis=0) * jnp.where(rowh < _HR - 1, 1.0, 0.0)

        me = even_col * (1.0 - psr)
        mo = (1.0 - even_col) * (1.0 - jnp.maximum(pf, pdn))
        mask_ref[0, 0, :, 0:_W] = me
        mask_ref[0, 0, :, _W:] = mo
        val_ref[0, 0, :, 0:_W] = psl
        val_ref[0, 0, :, _W:] = jnp.zeros((_HR, _W), jnp.float32)


@jax.jit
def kernel(sigma):
    sig4 = sigma.reshape(_B, _C, _HR, 2 * _W)
    out_sds = jax.ShapeDtypeStruct((_B, 1, _HR, 2 * _W), jnp.float32)

    def _mk_spec(i):
        return pl.BlockSpec((1, _CS, _BR, _W), lambda b, r: (b, i, r, 0))

    mask, values = pl.pallas_call(
        _body,
        grid=(_B, _NR),
        in_specs=[_mk_spec(i) for i in range(_NSPLIT)],
        out_specs=[
            pl.BlockSpec((1, 1, _HR, 2 * _W), lambda b, r: (b, 0, 0, 0)),
            pl.BlockSpec((1, 1, _HR, 2 * _W), lambda b, r: (b, 0, 0, 0)),
        ],
        out_shape=[out_sds, out_sds],
        scratch_shapes=[pltpu.VMEM((_HR, _W), jnp.float32)],
        compiler_params=pltpu.CompilerParams(
            dimension_semantics=("arbitrary", "arbitrary"),
        ),
    )(*([sig4] * _NSPLIT))
    return mask.reshape(_B, 1, _H, _W), values.reshape(_B, 1, _H, _W)


# speedup vs baseline: 2.0121x; 1.7810x over previous
import functools

import numpy as np
import jax
import jax.numpy as jnp
from jax import lax
from jax.experimental import pallas as pl
from jax.experimental.pallas import tpu as pltpu
from jax.experimental.pallas import tpu_sc as plsc

_THR = 0.18
_B, _C, _H, _W = 2, 96, 512, 512
_HR = _H // 2
_NTILES = 32
_RPT = _HR // _NTILES
_NGR = _RPT + 1
_CPC = 8
_NCH = _C // _CPC
_IPC = _CPC * _NGR
_NV = _W // 16


def _build_indices():
    wid = np.arange(_NTILES)[:, None, None, None]
    b = np.arange(_B)[None, :, None, None]
    chunk = np.arange(_NCH)[None, None, :, None]
    t = np.arange(_IPC)[None, None, None, :]
    ch = chunk * _CPC + t // _NGR
    r = np.minimum(wid * _RPT + t % _NGR, _HR - 1)
    row_id = (b * _C + ch) * _H + 2 * r
    return jnp.asarray(row_id.astype(np.int32))


def _sc_body(table, idxs, mask_out, val_out,
             idx_v, buf0, buf1, acc, p_buf, mtile, vtile, sem0, sem1):
    wid = lax.axis_index("c") * 16 + lax.axis_index("s")
    r0 = wid * _RPT

    io = lax.iota(jnp.int32, 16)
    one = jnp.full((16,), 1.0, jnp.float32)
    zero = jnp.full((16,), 0.0, jnp.float32)
    even_f = jnp.where(io % 2 == 0, 1.0, 0.0)
    odd_f = one - even_f
    pat1_f = jnp.where(io % 4 == 1, 1.0, 0.0)
    pat3_f = jnp.where(io % 4 == 3, 1.0, 0.0)
    idx_m1 = (io + 15) % 16
    idx_p1 = (io + 1) % 16
    inv_c = jnp.float32(1.0 / _C)

    _gdn = lax.GatherDimensionNumbers(
        offset_dims=(), collapsed_slice_dims=(0,), start_index_map=(0,))

    def _take(v, idx):
        return lax.gather(v, idx[:, None], _gdn, (1,),
                          mode=lax.GatherScatterMode.PROMISE_IN_BOUNDS)

    pltpu.sync_copy(idxs.at[wid], idx_v)

    bufs = (buf0, buf1)
    sems = (sem0, sem1)

    def gather(gi):
        b, chunk = divmod(gi, _NCH)
        return pltpu.make_async_copy(
            table.at[idx_v.at[b, chunk]], bufs[gi % 2], sems[gi % 2])

    def accum_chunk(buf, first):
        def rowloop(k, _):
            def colloop(j, _):
                base = j * 16
                v = buf[0 * _NGR + k, pl.ds(base, 16)]
                for ch in range(1, _CPC):
                    v = v + buf[ch * _NGR + k, pl.ds(base, 16)]
                if not first:
                    v = v + acc[k, pl.ds(base, 16)]
                acc[k, pl.ds(base, 16)] = v
                return 0
            return lax.fori_loop(0, _NV, colloop, 0, unroll=2)
        lax.fori_loop(0, _NGR, rowloop, 0)

    def emit_outputs(b):
        def prow(k, _):
            rr = r0 + k
            par = (rr % 2).astype(jnp.float32)
            patt = pat1_f + (pat3_f - pat1_f) * par

            def pcol(j, _):
                base = j * 16
                hm = acc[k, pl.ds(base, 16)] * inv_c
                p_buf[pl.ds(k * _W + base, 16)] = jnp.where(hm <= _THR, patt, zero)
                return 0
            return lax.fori_loop(0, _NV, pcol, 0, unroll=2)
        lax.fori_loop(0, _NGR, prow, 0)

        @pl.when(wid == _NTILES - 1)
        def _():
            def zcol(j, _):
                p_buf[pl.ds((_NGR - 1) * _W + j * 16, 16)] = zero
                return 0
            lax.fori_loop(0, _NV, zcol, 0)

        def orow(k, _):
            def ocol(j, carry):
                prev, cur = carry
                base = j * 16
                flat = k * _W + base
                last = (j == _NV - 1).astype(jnp.float32)
                nxt = p_buf[pl.ds(flat + 16, 16)] * (1.0 - last)
                pdn = p_buf[pl.ds(flat + _W, 16)]
                psr = jnp.where(io == 0, _take(prev, idx_m1), _take(cur, idx_m1))
                psl = jnp.where(io == 15, _take(nxt, idx_p1), _take(cur, idx_p1))
                mtile[2 * k, pl.ds(base, 16)] = even_f * (one - psr)
                mtile[2 * k + 1, pl.ds(base, 16)] = odd_f * (one - jnp.maximum(cur, pdn))
                vtile[2 * k, pl.ds(base, 16)] = even_f * psl
                vtile[2 * k + 1, pl.ds(base, 16)] = zero
                return (cur, nxt)
            cur0 = p_buf[pl.ds(k * _W, 16)]
            lax.fori_loop(0, _NV, ocol, (zero, cur0), unroll=2)
            return 0
        lax.fori_loop(0, _RPT, orow, 0)

        rows = pl.ds(wid * 2 * _RPT, 2 * _RPT)
        pltpu.sync_copy(mtile, mask_out.at[b, 0, rows, :])
        pltpu.sync_copy(vtile, val_out.at[b, 0, rows, :])

    gather(0).start()
    for gi in range(_B * _NCH):
        if gi + 1 < _B * _NCH:
            gather(gi + 1).start()
        gather(gi).wait()
        accum_chunk(bufs[gi % 2], first=(gi % _NCH == 0))
        if gi % _NCH == _NCH - 1:
            emit_outputs(gi // _NCH)


@jax.jit
def kernel(sigma):
    table = sigma.reshape(_B * _C * _H, _W)
    idxs = _build_indices()
    out_sds = jax.ShapeDtypeStruct((_B, 1, _H, _W), jnp.float32)
    mesh = plsc.VectorSubcoreMesh(core_axis_name="c", subcore_axis_name="s")
    sc_fn = functools.partial(
        pl.kernel,
        mesh=mesh,
        out_type=[out_sds, out_sds],
        scratch_types=[
            pltpu.VMEM((_B, _NCH, _IPC), jnp.int32),
            pltpu.VMEM((_IPC, _W), jnp.float32),
            pltpu.VMEM((_IPC, _W), jnp.float32),
            pltpu.VMEM((_NGR, _W), jnp.float32),
            pltpu.VMEM((_NGR * _W,), jnp.float32),
            pltpu.VMEM((2 * _RPT, _W), jnp.float32),
            pltpu.VMEM((2 * _RPT, _W), jnp.float32),
            pltpu.SemaphoreType.DMA,
            pltpu.SemaphoreType.DMA,
        ],
    )(_sc_body)
    mask, values = sc_fn(table, idxs)
    return mask, values


# speedup vs baseline: 3.1912x; 1.5860x over previous
import functools

import numpy as np
import jax
import jax.numpy as jnp
from jax import lax
from jax.experimental import pallas as pl
from jax.experimental.pallas import tpu as pltpu
from jax.experimental.pallas import tpu_sc as plsc

_THR = 0.18
_B, _C, _H, _W = 2, 96, 512, 512
_HR = _H // 2
_NTILES = 32
_RPT = _HR // _NTILES
_NGR = _RPT + 1
_CPC = 8
_NCH = _C // _CPC
_IPC = _CPC * _NGR
_NV = _W // 16


def _build_indices():
    wid = np.arange(_NTILES)[:, None, None, None]
    b = np.arange(_B)[None, :, None, None]
    chunk = np.arange(_NCH)[None, None, :, None]
    t = np.arange(_IPC)[None, None, None, :]
    ch = chunk * _CPC + t // _NGR
    r = np.minimum(wid * _RPT + t % _NGR, _HR - 1)
    row_id = (b * _C + ch) * _H + 2 * r
    return jnp.asarray(row_id.astype(np.int32))


def _sc_body(table, idxs, mask_out, val_out,
             idx_v, buf0, buf1, acc, p_buf, mtile, vtile, sem0, sem1):
    wid = lax.axis_index("c") * 16 + lax.axis_index("s")
    r0 = wid * _RPT

    io = lax.iota(jnp.int32, 16)
    one = jnp.full((16,), 1.0, jnp.float32)
    zero = jnp.full((16,), 0.0, jnp.float32)
    even_f = jnp.where(io % 2 == 0, 1.0, 0.0)
    odd_f = one - even_f
    pat1_f = jnp.where(io % 4 == 1, 1.0, 0.0)
    pat3_f = jnp.where(io % 4 == 3, 1.0, 0.0)
    idx_m1 = (io + 15) % 16
    idx_p1 = (io + 1) % 16
    inv_c = jnp.float32(1.0 / _C)

    _gdn = lax.GatherDimensionNumbers(
        offset_dims=(), collapsed_slice_dims=(0,), start_index_map=(0,))

    def _take(v, idx):
        return lax.gather(v, idx[:, None], _gdn, (1,),
                          mode=lax.GatherScatterMode.PROMISE_IN_BOUNDS)

    pltpu.sync_copy(idxs.at[wid], idx_v)

    bufs = (buf0, buf1)
    sems = (sem0, sem1)

    def gather(gi):
        b, chunk = divmod(gi, _NCH)
        return pltpu.make_async_copy(
            table.at[idx_v.at[b, chunk]], bufs[gi % 2], sems[gi % 2])

    def accum_chunk(buf, first):
        def rowloop(k, _):
            @plsc.parallel_loop(0, _NV, unroll=4)
            def colloop(j):
                base = j * 16
                v = buf[0 * _NGR + k, pl.ds(base, 16)]
                for ch in range(1, _CPC):
                    v = v + buf[ch * _NGR + k, pl.ds(base, 16)]
                if not first:
                    v = v + acc[k, pl.ds(base, 16)]
                acc[k, pl.ds(base, 16)] = v
            return 0
        lax.fori_loop(0, _NGR, rowloop, 0)

    def emit_outputs(b):
        def prow(k, _):
            rr = r0 + k
            par = (rr % 2).astype(jnp.float32)
            patt = pat1_f + (pat3_f - pat1_f) * par

            @plsc.parallel_loop(0, _NV, unroll=4)
            def pcol(j):
                base = j * 16
                hm = acc[k, pl.ds(base, 16)] * inv_c
                p_buf[pl.ds(k * _W + base, 16)] = jnp.where(hm <= _THR, patt, zero)
            return 0
        lax.fori_loop(0, _NGR, prow, 0)

        @pl.when(wid == _NTILES - 1)
        def _():
            def zcol(j, _):
                p_buf[pl.ds((_NGR - 1) * _W + j * 16, 16)] = zero
                return 0
            lax.fori_loop(0, _NV, zcol, 0)

        def orow(k, _):
            def ocol(j, carry):
                prev, cur = carry
                base = j * 16
                flat = k * _W + base
                last = (j == _NV - 1).astype(jnp.float32)
                nxt = p_buf[pl.ds(flat + 16, 16)] * (1.0 - last)
                pdn = p_buf[pl.ds(flat + _W, 16)]
                psr = jnp.where(io == 0, _take(prev, idx_m1), _take(cur, idx_m1))
                psl = jnp.where(io == 15, _take(nxt, idx_p1), _take(cur, idx_p1))
                mtile[2 * k, pl.ds(base, 16)] = even_f * (one - psr)
                mtile[2 * k + 1, pl.ds(base, 16)] = odd_f * (one - jnp.maximum(cur, pdn))
                vtile[2 * k, pl.ds(base, 16)] = even_f * psl
                vtile[2 * k + 1, pl.ds(base, 16)] = zero
                return (cur, nxt)
            cur0 = p_buf[pl.ds(k * _W, 16)]
            lax.fori_loop(0, _NV, ocol, (zero, cur0), unroll=2)
            return 0
        lax.fori_loop(0, _RPT, orow, 0)

        rows = pl.ds(wid * 2 * _RPT, 2 * _RPT)
        pltpu.sync_copy(mtile, mask_out.at[b, 0, rows, :])
        pltpu.sync_copy(vtile, val_out.at[b, 0, rows, :])

    gather(0).start()
    for gi in range(_B * _NCH):
        if gi + 1 < _B * _NCH:
            gather(gi + 1).start()
        gather(gi).wait()
        accum_chunk(bufs[gi % 2], first=(gi % _NCH == 0))
        if gi % _NCH == _NCH - 1:
            emit_outputs(gi // _NCH)


@jax.jit
def kernel(sigma):
    table = sigma.reshape(_B * _C * _H, _W)
    idxs = _build_indices()
    out_sds = jax.ShapeDtypeStruct((_B, 1, _H, _W), jnp.float32)
    mesh = plsc.VectorSubcoreMesh(core_axis_name="c", subcore_axis_name="s")
    sc_fn = functools.partial(
        pl.kernel,
        mesh=mesh,
        out_type=[out_sds, out_sds],
        scratch_types=[
            pltpu.VMEM((_B, _NCH, _IPC), jnp.int32),
            pltpu.VMEM((_IPC, _W), jnp.float32),
            pltpu.VMEM((_IPC, _W), jnp.float32),
            pltpu.VMEM((_NGR, _W), jnp.float32),
            pltpu.VMEM((_NGR * _W,), jnp.float32),
            pltpu.VMEM((2 * _RPT, _W), jnp.float32),
            pltpu.VMEM((2 * _RPT, _W), jnp.float32),
            pltpu.SemaphoreType.DMA,
            pltpu.SemaphoreType.DMA,
        ],
    )(_sc_body)
    mask, values = sc_fn(table, idxs)
    return mask, values


# speedup vs baseline: 3.2214x; 1.0095x over previous
import functools

import numpy as np
import jax
import jax.numpy as jnp
from jax import lax
from jax.experimental import pallas as pl
from jax.experimental.pallas import tpu as pltpu
from jax.experimental.pallas import tpu_sc as plsc

_THR = 0.18
_B, _C, _H, _W = 2, 96, 512, 512
_HR = _H // 2
_NTILES = 32
_RPT = _HR // _NTILES
_NGR = _RPT + 1
_CPC = 8
_NCH = _C // _CPC
_IPC = _CPC * _NGR
_NV = _W // 16


def _build_indices():
    wid = np.arange(_NTILES)[:, None, None, None]
    b = np.arange(_B)[None, :, None, None]
    chunk = np.arange(_NCH)[None, None, :, None]
    t = np.arange(_IPC)[None, None, None, :]
    ch = chunk * _CPC + t // _NGR
    r = np.minimum(wid * _RPT + t % _NGR, _HR - 1)
    row_id = (b * _C + ch) * _H + 2 * r
    return jnp.asarray(row_id.astype(np.int32))


def _sc_body(table, idxs, mask_out, val_out,
             idx_v, buf0, buf1, acc, p_buf, mtile, vtile, sem0, sem1):
    wid = lax.axis_index("c") * 16 + lax.axis_index("s")
    r0 = wid * _RPT

    io = lax.iota(jnp.int32, 16)
    one = jnp.full((16,), 1.0, jnp.float32)
    zero = jnp.full((16,), 0.0, jnp.float32)
    even_f = jnp.where(io % 2 == 0, 1.0, 0.0)
    odd_f = one - even_f
    pat1_f = jnp.where(io % 4 == 1, 1.0, 0.0)
    pat3_f = jnp.where(io % 4 == 3, 1.0, 0.0)
    idx_m1 = (io + 15) % 16
    idx_p1 = (io + 1) % 16
    inv_c = jnp.float32(1.0 / _C)

    _gdn = lax.GatherDimensionNumbers(
        offset_dims=(), collapsed_slice_dims=(0,), start_index_map=(0,))

    def _take(v, idx):
        return lax.gather(v, idx[:, None], _gdn, (1,),
                          mode=lax.GatherScatterMode.PROMISE_IN_BOUNDS)

    pltpu.sync_copy(idxs.at[wid], idx_v)

    bufs = (buf0, buf1)
    sems = (sem0, sem1)

    def gather(gi):
        b, chunk = divmod(gi, _NCH)
        return pltpu.make_async_copy(
            table.at[idx_v.at[b, chunk]], bufs[gi % 2], sems[gi % 2])

    def accum_chunk(buf, first):
        @plsc.parallel_loop(0, _NGR * _NV, unroll=8)
        def posloop(pos):
            k = pos // _NV
            base = (pos % _NV) * 16
            v = buf[0 * _NGR + k, pl.ds(base, 16)]
            for ch in range(1, _CPC):
                v = v + buf[ch * _NGR + k, pl.ds(base, 16)]
            if first:
                acc[k, pl.ds(base, 16)] = v
            else:
                plsc.addupdate(acc.at[k, pl.ds(base, 16)], v)

    def emit_outputs(b):
        def prow(k, _):
            rr = r0 + k
            par = (rr % 2).astype(jnp.float32)
            patt = pat1_f + (pat3_f - pat1_f) * par

            @plsc.parallel_loop(0, _NV, unroll=4)
            def pcol(j):
                base = j * 16
                hm = acc[k, pl.ds(base, 16)] * inv_c
                p_buf[pl.ds(k * _W + base, 16)] = jnp.where(hm <= _THR, patt, zero)
            return 0
        lax.fori_loop(0, _NGR, prow, 0)

        @pl.when(wid == _NTILES - 1)
        def _():
            def zcol(j, _):
                p_buf[pl.ds((_NGR - 1) * _W + j * 16, 16)] = zero
                return 0
            lax.fori_loop(0, _NV, zcol, 0)

        def orow(k, _):
            def ocol(j, carry):
                prev, cur = carry
                base = j * 16
                flat = k * _W + base
                last = (j == _NV - 1).astype(jnp.float32)
                nxt = p_buf[pl.ds(flat + 16, 16)] * (1.0 - last)
                pdn = p_buf[pl.ds(flat + _W, 16)]
                psr = jnp.where(io == 0, _take(prev, idx_m1), _take(cur, idx_m1))
                psl = jnp.where(io == 15, _take(nxt, idx_p1), _take(cur, idx_p1))
                mtile[2 * k, pl.ds(base, 16)] = even_f * (one - psr)
                mtile[2 * k + 1, pl.ds(base, 16)] = odd_f * (one - jnp.maximum(cur, pdn))
                vtile[2 * k, pl.ds(base, 16)] = even_f * psl
                vtile[2 * k + 1, pl.ds(base, 16)] = zero
                return (cur, nxt)
            cur0 = p_buf[pl.ds(k * _W, 16)]
            lax.fori_loop(0, _NV, ocol, (zero, cur0), unroll=2)
            return 0
        lax.fori_loop(0, _RPT, orow, 0)

        rows = pl.ds(wid * 2 * _RPT, 2 * _RPT)
        pltpu.sync_copy(mtile, mask_out.at[b, 0, rows, :])
        pltpu.sync_copy(vtile, val_out.at[b, 0, rows, :])

    gather(0).start()
    for gi in range(_B * _NCH):
        if gi + 1 < _B * _NCH:
            gather(gi + 1).start()
        gather(gi).wait()
        accum_chunk(bufs[gi % 2], first=(gi % _NCH == 0))
        if gi % _NCH == _NCH - 1:
            emit_outputs(gi // _NCH)


@jax.jit
def kernel(sigma):
    table = sigma.reshape(_B * _C * _H, _W)
    idxs = _build_indices()
    out_sds = jax.ShapeDtypeStruct((_B, 1, _H, _W), jnp.float32)
    mesh = plsc.VectorSubcoreMesh(core_axis_name="c", subcore_axis_name="s")
    sc_fn = functools.partial(
        pl.kernel,
        mesh=mesh,
        out_type=[out_sds, out_sds],
        scratch_types=[
            pltpu.VMEM((_B, _NCH, _IPC), jnp.int32),
            pltpu.VMEM((_IPC, _W), jnp.float32),
            pltpu.VMEM((_IPC, _W), jnp.float32),
            pltpu.VMEM((_NGR, _W), jnp.float32),
            pltpu.VMEM((_NGR * _W,), jnp.float32),
            pltpu.VMEM((2 * _RPT, _W), jnp.float32),
            pltpu.VMEM((2 * _RPT, _W), jnp.float32),
            pltpu.SemaphoreType.DMA,
            pltpu.SemaphoreType.DMA,
        ],
    )(_sc_body)
    mask, values = sc_fn(table, idxs)
    return mask, values


# speedup vs baseline: 3.4794x; 1.0801x over previous
import functools

import numpy as np
import jax
import jax.numpy as jnp
from jax import lax
from jax.experimental import pallas as pl
from jax.experimental.pallas import tpu as pltpu
from jax.experimental.pallas import tpu_sc as plsc

_THR = 0.18
_B, _C, _H, _W = 2, 96, 512, 512
_HR = _H // 2
_NSUB = 16
_RPT = _HR // _NSUB
_CPC = 4
_NCH = _C // _CPC
_IPC = _CPC * _RPT
_NV = _W // 16


def _build_indices():
    b = np.arange(_B)[:, None, None, None]
    sid = np.arange(_NSUB)[None, :, None, None]
    chunk = np.arange(_NCH)[None, None, :, None]
    t = np.arange(_IPC)[None, None, None, :]
    ch = chunk * _CPC + t // _RPT
    r = sid * _RPT + t % _RPT
    row_id = (b * _C + ch) * _H + 2 * r
    return jnp.asarray(row_id.astype(np.int32))


def _sc_body(table, idxs, mask_out, val_out,
             idx_v, buf0, buf1, acc, p_buf, mtile, vtile, shared, sem0, sem1):
    cid = lax.axis_index("c")
    sid = lax.axis_index("s")

    io = lax.iota(jnp.int32, 16)
    one = jnp.full((16,), 1.0, jnp.float32)
    zero = jnp.full((16,), 0.0, jnp.float32)
    even_f = jnp.where(io % 2 == 0, 1.0, 0.0)
    odd_f = one - even_f
    pat1_f = jnp.where(io % 4 == 1, 1.0, 0.0)
    pat3_f = jnp.where(io % 4 == 3, 1.0, 0.0)
    idx_m1 = (io + 15) % 16
    idx_p1 = (io + 1) % 16
    inv_c = jnp.float32(1.0 / _C)

    _gdn = lax.GatherDimensionNumbers(
        offset_dims=(), collapsed_slice_dims=(0,), start_index_map=(0,))

    def _take(v, idx):
        return lax.gather(v, idx[:, None], _gdn, (1,),
                          mode=lax.GatherScatterMode.PROMISE_IN_BOUNDS)

    pltpu.sync_copy(idxs.at[cid, sid], idx_v)

    bufs = (buf0, buf1)
    sems = (sem0, sem1)

    def gather(gi):
        return pltpu.make_async_copy(
            table.at[idx_v.at[gi]], bufs[gi % 2], sems[gi % 2])

    def accum_chunk(buf, first):
        @plsc.parallel_loop(0, _RPT * _NV, unroll=8)
        def posloop(pos):
            k = pos // _NV
            base = (pos % _NV) * 16
            v = buf[0 * _RPT + k, pl.ds(base, 16)]
            for ch in range(1, _CPC):
                v = v + buf[ch * _RPT + k, pl.ds(base, 16)]
            if first:
                acc[k, pl.ds(base, 16)] = v
            else:
                plsc.addupdate(acc.at[k, pl.ds(base, 16)], v)

    gather(0).start()
    for gi in range(_NCH):
        if gi + 1 < _NCH:
            gather(gi + 1).start()
        gather(gi).wait()
        accum_chunk(bufs[gi % 2], first=(gi == 0))

    def prow(k, _):
        rr = sid * _RPT + k
        par = (rr % 2).astype(jnp.float32)
        patt = pat1_f + (pat3_f - pat1_f) * par

        @plsc.parallel_loop(0, _NV, unroll=4)
        def pcol(j):
            base = j * 16
            hm = acc[k, pl.ds(base, 16)] * inv_c
            p_buf[pl.ds(k * _W + base, 16)] = jnp.where(hm <= _THR, patt, zero)
        return 0
    lax.fori_loop(0, _RPT, prow, 0)

    pltpu.sync_copy(p_buf.at[pl.ds(0, _W)], shared.at[sid])
    plsc.subcore_barrier()

    @pl.when(sid < _NSUB - 1)
    def _():
        pltpu.sync_copy(shared.at[sid + 1], p_buf.at[pl.ds(_RPT * _W, _W)])

    @pl.when(sid == _NSUB - 1)
    def _():
        @plsc.parallel_loop(0, _NV, unroll=4)
        def zcol(j):
            p_buf[pl.ds(_RPT * _W + j * 16, 16)] = zero

    def orow(k, _):
        def ocol(j, carry):
            prev, cur = carry
            base = j * 16
            flat = k * _W + base
            last = (j == _NV - 1).astype(jnp.float32)
            nxt = p_buf[pl.ds(flat + 16, 16)] * (1.0 - last)
            pdn = p_buf[pl.ds(flat + _W, 16)]
            psr = jnp.where(io == 0, _take(prev, idx_m1), _take(cur, idx_m1))
            psl = jnp.where(io == 15, _take(nxt, idx_p1), _take(cur, idx_p1))
            mtile[2 * k, pl.ds(base, 16)] = even_f * (one - psr)
            mtile[2 * k + 1, pl.ds(base, 16)] = odd_f * (one - jnp.maximum(cur, pdn))
            vtile[2 * k, pl.ds(base, 16)] = even_f * psl
            vtile[2 * k + 1, pl.ds(base, 16)] = zero
            return (cur, nxt)
        cur0 = p_buf[pl.ds(k * _W, 16)]
        lax.fori_loop(0, _NV, ocol, (zero, cur0), unroll=2)
        return 0
    lax.fori_loop(0, _RPT, orow, 0)

    rows = pl.ds(sid * 2 * _RPT, 2 * _RPT)
    pltpu.sync_copy(mtile, mask_out.at[cid, 0, rows, :])
    pltpu.sync_copy(vtile, val_out.at[cid, 0, rows, :])


@jax.jit
def kernel(sigma):
    table = sigma.reshape(_B * _C * _H, _W)
    idxs = _build_indices()
    out_sds = jax.ShapeDtypeStruct((_B, 1, _H, _W), jnp.float32)
    mesh = plsc.VectorSubcoreMesh(core_axis_name="c", subcore_axis_name="s")
    sc_fn = functools.partial(
        pl.kernel,
        mesh=mesh,
        out_type=[out_sds, out_sds],
        scratch_types=[
            pltpu.VMEM((_NCH, _IPC), jnp.int32),
            pltpu.VMEM((_IPC, _W), jnp.float32),
            pltpu.VMEM((_IPC, _W), jnp.float32),
            pltpu.VMEM((_RPT, _W), jnp.float32),
            pltpu.VMEM(((_RPT + 1) * _W,), jnp.float32),
            pltpu.VMEM((2 * _RPT, _W), jnp.float32),
            pltpu.VMEM((2 * _RPT, _W), jnp.float32),
            pltpu.VMEM_SHARED((_NSUB, _W), jnp.float32),
            pltpu.SemaphoreType.DMA,
            pltpu.SemaphoreType.DMA,
        ],
    )(_sc_body)
    mask, values = sc_fn(table, idxs)
    return mask, values
